# baseline (speedup 1.0000x reference)
import jax
import jax.numpy as jnp
from jax import lax
from jax.experimental import pallas as pl
from jax.experimental.pallas import tpu as pltpu
from jax.experimental.pallas import tpu_sc as plsc

EMBED_DIM = 1024
T_LEN = 8192
N_TILES = 32
LANES = 16
C_ROWS = 128
RPT = C_ROWS // N_TILES
GROUPS = EMBED_DIM // LANES
TC_BLK = 2048


def _add_window(dst, src_fn, n_rows, upg=16):
    per_row = GROUPS // upg
    def body(i, carry):
        r = lax.div(i, per_row)
        jb = lax.rem(i, per_row)
        for u in range(upg):
            sl = pl.ds(jb * (upg * LANES) + u * LANES, LANES)
            plsc.addupdate(dst.at[r, sl], src_fn(r, sl))
        return carry
    lax.fori_loop(0, n_rows * per_row, body, 0)


def _sc_body(bar_ref, qn_ref, bar8_ref, c_ref,
             cbuf, barbuf, qnbuf, sem0, sem1, sem2):
    cid = lax.axis_index("c")
    sid = lax.axis_index("s")
    wid = cid * 16 + sid
    row0 = wid * RPT

    d0 = pltpu.async_copy(bar8_ref.at[pl.ds(row0, RPT)], cbuf, sem0)
    d1 = pltpu.async_copy(bar_ref.at[pl.ds((wid % 4) * RPT, RPT)], barbuf,
                          sem1)
    d2 = pltpu.async_copy(qn_ref, qnbuf, sem2)
    d0.wait()
    d1.wait()
    d2.wait()
    _add_window(cbuf, lambda r, sl: barbuf[r, sl] + qnbuf[r, sl], RPT)
    pltpu.sync_copy(cbuf, c_ref.at[pl.ds(row0, RPT)])


TC_COL = 512


def _tc_body(c_ref, glob_ref, out_ref):
    c = c_ref[...]
    for j in range(TC_BLK // C_ROWS):
        sl = pl.ds(j * C_ROWS, C_ROWS)
        out_ref[sl, :] = glob_ref[sl, :] + c


def kernel(x, bar_w, qn_w, bar8_w, global_w):
    del x

    mesh = plsc.VectorSubcoreMesh(core_axis_name="c", subcore_axis_name="s",
                                  num_cores=2, num_subcores=16)
    sc_fn = pl.kernel(
        _sc_body,
        out_type=jax.ShapeDtypeStruct((C_ROWS, EMBED_DIM), jnp.float32),
        mesh=mesh,
        scratch_types=[
            pltpu.VMEM((RPT, EMBED_DIM), jnp.float32),
            pltpu.VMEM((RPT, EMBED_DIM), jnp.float32),
            pltpu.VMEM((4, EMBED_DIM), jnp.float32),
            pltpu.SemaphoreType.DMA,
            pltpu.SemaphoreType.DMA,
            pltpu.SemaphoreType.DMA,
        ],
    )
    c = sc_fn(bar_w, qn_w, bar8_w)

    pe = pl.pallas_call(
        _tc_body,
        grid=(T_LEN // TC_BLK, EMBED_DIM // TC_COL),
        in_specs=[
            pl.BlockSpec((C_ROWS, TC_COL), lambda i, j: (0, j)),
            pl.BlockSpec((TC_BLK, TC_COL), lambda i, j: (i, j)),
        ],
        out_specs=pl.BlockSpec((TC_BLK, TC_COL), lambda i, j: (i, j)),
        out_shape=jax.ShapeDtypeStruct((T_LEN, EMBED_DIM), jnp.float32),
    )(c, global_w)

    return pe[None, :, :]


# speedup vs baseline: 1.0495x; 1.0495x over previous
import jax
import jax.numpy as jnp
from jax import lax
from jax.experimental import pallas as pl
from jax.experimental.pallas import tpu as pltpu
from jax.experimental.pallas import tpu_sc as plsc

EMBED_DIM = 1024
T_LEN = 8192
N_TILES = 32
LANES = 16
C_ROWS = 128
RPT = C_ROWS // N_TILES
GROUPS = EMBED_DIM // LANES
TC_BLK = 2048


def _add_window(dst, src_fn, n_rows, upg=16):
    per_row = GROUPS // upg
    def body(i, carry):
        r = lax.div(i, per_row)
        jb = lax.rem(i, per_row)
        for u in range(upg):
            sl = pl.ds(jb * (upg * LANES) + u * LANES, LANES)
            plsc.addupdate(dst.at[r, sl], src_fn(r, sl))
        return carry
    lax.fori_loop(0, n_rows * per_row, body, 0)


def _sc_body(bar_ref, qn_ref, bar8_ref, c_ref,
             cbuf, barbuf, qnbuf, sem0, sem1, sem2):
    cid = lax.axis_index("c")
    sid = lax.axis_index("s")
    wid = cid * 16 + sid
    row0 = wid * RPT

    d0 = pltpu.async_copy(bar8_ref.at[pl.ds(row0, RPT)], cbuf, sem0)
    d1 = pltpu.async_copy(bar_ref.at[pl.ds((wid % 4) * RPT, RPT)], barbuf,
                          sem1)
    d2 = pltpu.async_copy(qn_ref, qnbuf, sem2)
    d0.wait()
    d1.wait()
    d2.wait()
    _add_window(cbuf, lambda r, sl: barbuf[r, sl] + qnbuf[r, sl], RPT)
    pltpu.sync_copy(cbuf, c_ref.at[pl.ds(row0, RPT)])


def _tc_body(c_ref, glob_ref, out_ref):
    c = c_ref[...]
    for j in range(TC_BLK // C_ROWS):
        sl = pl.ds(j * C_ROWS, C_ROWS)
        out_ref[sl, :] = glob_ref[sl, :] + c


def kernel(x, bar_w, qn_w, bar8_w, global_w):
    del x

    mesh = plsc.VectorSubcoreMesh(core_axis_name="c", subcore_axis_name="s",
                                  num_cores=2, num_subcores=16)
    sc_fn = pl.kernel(
        _sc_body,
        out_type=jax.ShapeDtypeStruct((C_ROWS, EMBED_DIM), jnp.float32),
        mesh=mesh,
        scratch_types=[
            pltpu.VMEM((RPT, EMBED_DIM), jnp.float32),
            pltpu.VMEM((RPT, EMBED_DIM), jnp.float32),
            pltpu.VMEM((4, EMBED_DIM), jnp.float32),
            pltpu.SemaphoreType.DMA,
            pltpu.SemaphoreType.DMA,
            pltpu.SemaphoreType.DMA,
        ],
    )
    c = sc_fn(bar_w, qn_w, bar8_w)

    pe = pl.pallas_call(
        _tc_body,
        grid=(T_LEN // TC_BLK,),
        in_specs=[
            pl.BlockSpec((C_ROWS, EMBED_DIM), lambda i: (0, 0)),
            pl.BlockSpec((TC_BLK, EMBED_DIM), lambda i: (i, 0)),
        ],
        out_specs=pl.BlockSpec((TC_BLK, EMBED_DIM), lambda i: (i, 0)),
        out_shape=jax.ShapeDtypeStruct((T_LEN, EMBED_DIM), jnp.float32),
    )(c, global_w)

    return pe[None, :, :]
